# Initial kernel scaffold; baseline (speedup 1.0000x reference)
import jax
import jax.numpy as jnp
from jax.experimental import pallas as pl


def kernel(H, batch_idx):
    raise NotImplementedError("write your pallas kernel here")


# speedup vs baseline: 6.3432x; 6.3432x over previous
import functools

import jax
import jax.numpy as jnp
from jax import lax
from jax.experimental import pallas as pl
from jax.experimental.pallas import tpu as pltpu
from jax.experimental.pallas import tpu_sc as plsc

_NSEG = 512
_D = 128
_N = 320000
_NC = 2
_NS = 16
_NW = _NC * _NS
_ROWS_W = _N // _NW
_BLK = 80
_NBLK = _ROWS_W // _BLK
_NBUF = 5

_mesh = plsc.VectorSubcoreMesh(core_axis_name="c", subcore_axis_name="s")


@functools.partial(
    pl.kernel,
    out_type=jax.ShapeDtypeStruct((_NC, _NSEG, _D), jnp.float32),
    mesh=_mesh,
    scratch_types=[
        pltpu.VMEM((_NBLK, _BLK), jnp.int32),
        pltpu.VMEM((_NBUF, _BLK, _D), jnp.float32),
        pltpu.VMEM_SHARED((_NSEG, _D), jnp.float32),
        [pltpu.SemaphoreType.DMA] * _NBUF,
    ],
)
def _seg_sum_sc(h_hbm, idx_hbm, zeros_hbm, out_hbm, idx_v, buf, acc, sems):
    cid = lax.axis_index("c")
    sid = lax.axis_index("s")
    wid = cid * _NS + sid
    base = wid * _ROWS_W

    stripe = _NSEG // _NS
    pltpu.sync_copy(zeros_hbm.at[pl.ds(sid * stripe, stripe)],
                    acc.at[pl.ds(sid * stripe, stripe)])

    pltpu.sync_copy(idx_hbm.at[wid], idx_v)
    plsc.subcore_barrier()

    for b in range(_NBUF):
        pltpu.async_copy(h_hbm.at[pl.ds(base + b * _BLK, _BLK)],
                         buf.at[b], sems[b])

    def body(i, carry):
        j = i * _NBUF
        for b in range(_NBUF):
            jj = j + b
            pltpu.make_async_copy(
                h_hbm.at[pl.ds(base, _BLK)], buf.at[b], sems[b]).wait()
            pltpu.sync_copy(buf.at[b], acc.at[idx_v.at[jj]], add=True)

            @pl.when(jj + _NBUF < _NBLK)
            def _prefetch():
                pltpu.async_copy(
                    h_hbm.at[pl.ds(base + (jj + _NBUF) * _BLK, _BLK)],
                    buf.at[b], sems[b])
        return carry

    lax.fori_loop(0, _NBLK // _NBUF, body, 0)

    plsc.subcore_barrier()

    @pl.when(sid == 0)
    def _writeback():
        pltpu.sync_copy(acc, out_hbm.at[cid])


def _sum2_body(p_ref, o_ref):
    o_ref[...] = p_ref[0] + p_ref[1]


_sum2_tc = pl.pallas_call(
    _sum2_body,
    out_shape=jax.ShapeDtypeStruct((_NSEG, _D), jnp.float32),
)


def kernel(H, batch_idx):
    idx = batch_idx.astype(jnp.int32).reshape(_NW, _NBLK, _BLK)
    zeros = jnp.zeros((_NSEG, _D), jnp.float32)
    partials = _seg_sum_sc(H, idx, zeros)
    return _sum2_tc(partials)
